# Initial kernel scaffold; baseline (speedup 1.0000x reference)
import jax
import jax.numpy as jnp
from jax.experimental import pallas as pl


def kernel(x, positions, kv_cache, block_tables, context_lens, slot_mapping, wq_a, q_norm_w, wq_b, wkv, kv_norm_w, wo_a, wo_b, attn_sink):
    raise NotImplementedError("write your pallas kernel here")


# speedup vs baseline: 12.7723x; 12.7723x over previous
import jax
import jax.numpy as jnp
import numpy as np
from jax.experimental import pallas as pl
from jax.experimental.pallas import tpu as pltpu

DIM = 2048
N_HEADS = 16
HEAD_DIM = 192
ROPE_DIM = 64
NOPE_DIM = HEAD_DIM - ROPE_DIM
Q_LORA = 1024
O_LORA = 512
N_GROUPS = 4
BLOCK_SIZE = 256
MAX_LEN = 8192
NUM_BLOCKS = 2048
TOTAL_SLOTS = NUM_BLOCKS * BLOCK_SIZE
N_TOK = 64
EPS = 1e-6
ROPE_THETA = 10000.0
SCALE = HEAD_DIM ** -0.5
NUM_PAGES = MAX_LEN // BLOCK_SIZE
QDIM = N_HEADS * HEAD_DIM
NEG_INF = -1e30


def _rope_patterns(width, n_heads):
    freq = np.zeros((width,), np.float32)
    sign = np.zeros((width,), np.float32)
    for h in range(n_heads):
        base = h * HEAD_DIM + NOPE_DIM
        for p in range(ROPE_DIM // 2):
            f = ROPE_THETA ** (-(2.0 * p) / ROPE_DIM)
            freq[base + 2 * p] = f
            freq[base + 2 * p + 1] = f
            sign[base + 2 * p] = -1.0
            sign[base + 2 * p + 1] = 1.0
    return freq, sign


_Q_FREQ, _Q_SIGN = _rope_patterns(QDIM, N_HEADS)
_KV_FREQ, _KV_SIGN = _rope_patterns(HEAD_DIM, 1)


def _apply_rope(v, pos_col, freq_row, sign_row):
    ang = pos_col * freq_row
    c = jnp.cos(ang)
    s = jnp.sin(ang) * sign_row
    even = (jax.lax.broadcasted_iota(jnp.int32, v.shape, 1) % 2) == 0
    w = v.shape[1]
    swapped = jnp.where(even, pltpu.roll(v, w - 1, 1), pltpu.roll(v, 1, 1))
    return v * c + swapped * s


def _proj_body(x_ref, pos_ref, wqa_ref, qnw_ref, wqb_ref, wkv_ref, kvnw_ref,
               qfreq_ref, qsign_ref, kfreq_ref, ksign_ref, q_out, kv_out):
    x = x_ref[...]
    pos = pos_ref[...].astype(jnp.float32)

    ql = jax.lax.dot_general(x, wqa_ref[...], (((1,), (1,)), ((), ())),
                             preferred_element_type=jnp.float32)
    ql = ql * jax.lax.rsqrt(jnp.mean(ql * ql, axis=-1, keepdims=True) + EPS)
    ql = ql * qnw_ref[...]
    q = jax.lax.dot_general(ql, wqb_ref[...], (((1,), (1,)), ((), ())),
                            preferred_element_type=jnp.float32)
    q_out[...] = _apply_rope(q, pos, qfreq_ref[...], qsign_ref[...])

    kv = jax.lax.dot_general(x, wkv_ref[...], (((1,), (1,)), ((), ())),
                             preferred_element_type=jnp.float32)
    kv = kv * jax.lax.rsqrt(jnp.mean(kv * kv, axis=-1, keepdims=True) + EPS)
    kv = kv * kvnw_ref[...]
    kv_out[...] = _apply_rope(kv, pos, kfreq_ref[...], ksign_ref[...])


def _attn_body(bt_ref, cl_ref, q_ref, cache_ref, newkv_ref, slotmap_ref,
               sink_ref, out_ref, acc_ref, m_ref, l_ref):
    i = pl.program_id(0)
    j = pl.program_id(1)
    ctx = cl_ref[i]

    @pl.when(j == 0)
    def _init():
        m_ref[...] = sink_ref[...]
        l_ref[...] = jnp.ones_like(l_ref)
        acc_ref[...] = jnp.zeros_like(acc_ref)

    @pl.when(j * BLOCK_SIZE < ctx)
    def _step():
        k = cache_ref[0]
        bid = bt_ref[i, j]
        row_ids = bid * BLOCK_SIZE + jax.lax.broadcasted_iota(
            jnp.int32, (BLOCK_SIZE, 1), 0)
        hits = row_ids == slotmap_ref[...]
        order = jax.lax.broadcasted_iota(jnp.int32, (BLOCK_SIZE, N_TOK), 1) + 1
        w = jnp.where(hits, order, 0)
        best = jnp.max(w, axis=1, keepdims=True)
        sel = (w == best) & (best > 0)
        patch = jax.lax.dot_general(
            sel.astype(jnp.float32), newkv_ref[...],
            (((1,), (0,)), ((), ())), preferred_element_type=jnp.float32)
        k = jnp.where(best > 0, patch, k)

        q = q_ref[0]
        s = jax.lax.dot_general(q, k, (((1,), (1,)), ((), ())),
                                preferred_element_type=jnp.float32) * SCALE
        pos = j * BLOCK_SIZE + jax.lax.broadcasted_iota(
            jnp.int32, (1, BLOCK_SIZE), 1)
        s = jnp.where(pos < ctx, s, NEG_INF)

        m_prev = m_ref[...]
        m_new = jnp.maximum(m_prev, jnp.max(s, axis=1, keepdims=True))
        alpha = jnp.exp(m_prev - m_new)
        p = jnp.exp(s - m_new)
        l_ref[...] = l_ref[...] * alpha + jnp.sum(p, axis=1, keepdims=True)
        acc_ref[...] = acc_ref[...] * alpha + jax.lax.dot_general(
            p, k, (((1,), (0,)), ((), ())), preferred_element_type=jnp.float32)
        m_ref[...] = m_new

    @pl.when(j == NUM_PAGES - 1)
    def _fin():
        out_ref[0] = acc_ref[...] / l_ref[...]


def _out_body(a_ref, woa_ref, wob_ref, o_ref):
    a = a_ref[...]
    gdim = QDIM // N_GROUPS
    parts = []
    for g in range(N_GROUPS):
        ag = a[:, g * gdim:(g + 1) * gdim]
        parts.append(jax.lax.dot_general(
            ag, woa_ref[g], (((1,), (1,)), ((), ())),
            preferred_element_type=jnp.float32))
    og = jnp.concatenate(parts, axis=1)
    o_ref[...] = jax.lax.dot_general(og, wob_ref[...], (((1,), (1,)), ((), ())),
                                     preferred_element_type=jnp.float32)


def kernel(x, positions, kv_cache, block_tables, context_lens, slot_mapping,
           wq_a, q_norm_w, wq_b, wkv, kv_norm_w, wo_a, wo_b, attn_sink):
    n = x.shape[0]
    pos_col = positions.reshape(n, 1).astype(jnp.int32)

    q, new_kv = pl.pallas_call(
        _proj_body,
        out_shape=(
            jax.ShapeDtypeStruct((n, QDIM), jnp.float32),
            jax.ShapeDtypeStruct((n, HEAD_DIM), jnp.float32),
        ),
    )(x, pos_col, wq_a, q_norm_w.reshape(1, Q_LORA), wq_b, wkv,
      kv_norm_w.reshape(1, HEAD_DIM),
      jnp.asarray(_Q_FREQ).reshape(1, QDIM),
      jnp.asarray(_Q_SIGN).reshape(1, QDIM),
      jnp.asarray(_KV_FREQ).reshape(1, HEAD_DIM),
      jnp.asarray(_KV_SIGN).reshape(1, HEAD_DIM))

    q = q.reshape(n, N_HEADS, HEAD_DIM)
    cache = kv_cache.reshape(NUM_BLOCKS, BLOCK_SIZE, HEAD_DIM)

    attn = pl.pallas_call(
        _attn_body,
        grid_spec=pltpu.PrefetchScalarGridSpec(
            num_scalar_prefetch=2,
            grid=(n, NUM_PAGES),
            in_specs=[
                pl.BlockSpec((1, N_HEADS, HEAD_DIM),
                             lambda i, j, bt, cl: (i, 0, 0)),
                pl.BlockSpec((1, BLOCK_SIZE, HEAD_DIM),
                             lambda i, j, bt, cl: (
                                 bt[i, jnp.minimum(
                                     j, (cl[i] - 1) // BLOCK_SIZE)], 0, 0)),
                pl.BlockSpec((N_TOK, HEAD_DIM), lambda i, j, bt, cl: (0, 0)),
                pl.BlockSpec((1, N_TOK), lambda i, j, bt, cl: (0, 0)),
                pl.BlockSpec((N_HEADS, 1), lambda i, j, bt, cl: (0, 0)),
            ],
            out_specs=pl.BlockSpec((1, N_HEADS, HEAD_DIM),
                                   lambda i, j, bt, cl: (i, 0, 0)),
            scratch_shapes=[
                pltpu.VMEM((N_HEADS, HEAD_DIM), jnp.float32),
                pltpu.VMEM((N_HEADS, 1), jnp.float32),
                pltpu.VMEM((N_HEADS, 1), jnp.float32),
            ],
        ),
        out_shape=jax.ShapeDtypeStruct((n, N_HEADS, HEAD_DIM), jnp.float32),
        compiler_params=pltpu.CompilerParams(
            dimension_semantics=("arbitrary", "arbitrary")),
    )(block_tables.astype(jnp.int32), context_lens.astype(jnp.int32),
      q, cache, new_kv, slot_mapping.reshape(1, n).astype(jnp.int32),
      attn_sink.reshape(N_HEADS, 1))

    o = pl.pallas_call(
        _out_body,
        out_shape=jax.ShapeDtypeStruct((n, DIM), jnp.float32),
    )(attn.reshape(n, QDIM), wo_a, wo_b)
    return o


# speedup vs baseline: 17.0791x; 1.3372x over previous
import jax
import jax.numpy as jnp
import numpy as np
from jax.experimental import pallas as pl
from jax.experimental.pallas import tpu as pltpu

DIM = 2048
N_HEADS = 16
HEAD_DIM = 192
ROPE_DIM = 64
NOPE_DIM = HEAD_DIM - ROPE_DIM
Q_LORA = 1024
O_LORA = 512
N_GROUPS = 4
BLOCK_SIZE = 256
MAX_LEN = 8192
NUM_BLOCKS = 2048
TOTAL_SLOTS = NUM_BLOCKS * BLOCK_SIZE
N_TOK = 64
EPS = 1e-6
ROPE_THETA = 10000.0
SCALE = HEAD_DIM ** -0.5
NUM_PAGES = MAX_LEN // BLOCK_SIZE
QDIM = N_HEADS * HEAD_DIM
NEG_INF = -1e30


def _rope_patterns(width, n_heads):
    freq = np.zeros((width,), np.float32)
    sign = np.zeros((width,), np.float32)
    for h in range(n_heads):
        base = h * HEAD_DIM + NOPE_DIM
        for p in range(ROPE_DIM // 2):
            f = ROPE_THETA ** (-(2.0 * p) / ROPE_DIM)
            freq[base + 2 * p] = f
            freq[base + 2 * p + 1] = f
            sign[base + 2 * p] = -1.0
            sign[base + 2 * p + 1] = 1.0
    return freq, sign


_Q_FREQ, _Q_SIGN = _rope_patterns(QDIM, N_HEADS)
_KV_FREQ, _KV_SIGN = _rope_patterns(HEAD_DIM, 1)


def _apply_rope(v, pos_col, freq_row, sign_row):
    ang = pos_col * freq_row
    c = jnp.cos(ang)
    s = jnp.sin(ang) * sign_row
    even = (jax.lax.broadcasted_iota(jnp.int32, v.shape, 1) % 2) == 0
    w = v.shape[1]
    swapped = jnp.where(even, pltpu.roll(v, w - 1, 1), pltpu.roll(v, 1, 1))
    return v * c + swapped * s


def _proj_body(x_ref, pos_ref, wqa_ref, qnw_ref, wqb_ref, wkv_ref, kvnw_ref,
               qfreq_ref, qsign_ref, kfreq_ref, ksign_ref, q_out, kv_out):
    x = x_ref[...]
    pos = pos_ref[...].astype(jnp.float32)

    ql = jax.lax.dot_general(x, wqa_ref[...], (((1,), (1,)), ((), ())),
                             preferred_element_type=jnp.float32)
    ql = ql * jax.lax.rsqrt(jnp.mean(ql * ql, axis=-1, keepdims=True) + EPS)
    ql = ql * qnw_ref[...]
    q = jax.lax.dot_general(ql, wqb_ref[...], (((1,), (1,)), ((), ())),
                            preferred_element_type=jnp.float32)
    q_out[...] = _apply_rope(q, pos, qfreq_ref[...], qsign_ref[...])

    kv = jax.lax.dot_general(x, wkv_ref[...], (((1,), (1,)), ((), ())),
                             preferred_element_type=jnp.float32)
    kv = kv * jax.lax.rsqrt(jnp.mean(kv * kv, axis=-1, keepdims=True) + EPS)
    kv = kv * kvnw_ref[...]
    kv_out[...] = _apply_rope(kv, pos, kfreq_ref[...], ksign_ref[...])


def _attn_body(bt_ref, cl_ref, hit_ref, q_ref, cache_ref, newkv_ref,
               slotmap_ref, sink_ref, out_ref, acc_ref, m_ref, l_ref):
    i = pl.program_id(0)
    j = pl.program_id(1)
    ctx = cl_ref[i]

    @pl.when(j == 0)
    def _init():
        m_ref[...] = sink_ref[...]
        l_ref[...] = jnp.ones_like(l_ref)
        acc_ref[...] = jnp.zeros_like(acc_ref)

    @pl.when(j * BLOCK_SIZE < ctx)
    def _step():
        @pl.when(hit_ref[i, j] != 0)
        def _patch():
            k0 = cache_ref[0]
            bid = bt_ref[i, j]
            row_ids = bid * BLOCK_SIZE + jax.lax.broadcasted_iota(
                jnp.int32, (BLOCK_SIZE, 1), 0)
            hits = row_ids == slotmap_ref[...]
            order = jax.lax.broadcasted_iota(
                jnp.int32, (BLOCK_SIZE, N_TOK), 1) + 1
            w = jnp.where(hits, order, 0)
            best = jnp.max(w, axis=1, keepdims=True)
            sel = (w == best) & (best > 0)
            patch = jax.lax.dot_general(
                sel.astype(jnp.float32), newkv_ref[...],
                (((1,), (0,)), ((), ())), preferred_element_type=jnp.float32)
            cache_ref[0] = jnp.where(best > 0, patch, k0)

        k = cache_ref[0]
        q = q_ref[0]
        s = jax.lax.dot_general(q, k, (((1,), (1,)), ((), ())),
                                preferred_element_type=jnp.float32) * SCALE
        pos = j * BLOCK_SIZE + jax.lax.broadcasted_iota(
            jnp.int32, (1, BLOCK_SIZE), 1)
        s = jnp.where(pos < ctx, s, NEG_INF)

        m_prev = m_ref[...]
        m_new = jnp.maximum(m_prev, jnp.max(s, axis=1, keepdims=True))
        alpha = jnp.exp(m_prev - m_new)
        p = jnp.exp(s - m_new)
        l_ref[...] = l_ref[...] * alpha + jnp.sum(p, axis=1, keepdims=True)
        acc_ref[...] = acc_ref[...] * alpha + jax.lax.dot_general(
            p, k, (((1,), (0,)), ((), ())), preferred_element_type=jnp.float32)
        m_ref[...] = m_new

    @pl.when(j == NUM_PAGES - 1)
    def _fin():
        out_ref[0] = acc_ref[...] / l_ref[...]


def _out_body(a_ref, woa_ref, wob_ref, o_ref):
    a = a_ref[...]
    gdim = QDIM // N_GROUPS
    parts = []
    for g in range(N_GROUPS):
        ag = a[:, g * gdim:(g + 1) * gdim]
        parts.append(jax.lax.dot_general(
            ag, woa_ref[g], (((1,), (1,)), ((), ())),
            preferred_element_type=jnp.float32))
    og = jnp.concatenate(parts, axis=1)
    o_ref[...] = jax.lax.dot_general(og, wob_ref[...], (((1,), (1,)), ((), ())),
                                     preferred_element_type=jnp.float32)


def kernel(x, positions, kv_cache, block_tables, context_lens, slot_mapping,
           wq_a, q_norm_w, wq_b, wkv, kv_norm_w, wo_a, wo_b, attn_sink):
    n = x.shape[0]
    pos_col = positions.reshape(n, 1).astype(jnp.int32)

    q, new_kv = pl.pallas_call(
        _proj_body,
        out_shape=(
            jax.ShapeDtypeStruct((n, QDIM), jnp.float32),
            jax.ShapeDtypeStruct((n, HEAD_DIM), jnp.float32),
        ),
    )(x, pos_col, wq_a, q_norm_w.reshape(1, Q_LORA), wq_b, wkv,
      kv_norm_w.reshape(1, HEAD_DIM),
      jnp.asarray(_Q_FREQ).reshape(1, QDIM),
      jnp.asarray(_Q_SIGN).reshape(1, QDIM),
      jnp.asarray(_KV_FREQ).reshape(1, HEAD_DIM),
      jnp.asarray(_KV_SIGN).reshape(1, HEAD_DIM))

    q = q.reshape(n, N_HEADS, HEAD_DIM)
    cache = kv_cache.reshape(NUM_BLOCKS, BLOCK_SIZE, HEAD_DIM)

    bt32 = block_tables.astype(jnp.int32)
    written_pages = (slot_mapping.astype(jnp.int32) // BLOCK_SIZE)
    hit_flags = jnp.any(
        bt32[:, :, None] == written_pages[None, None, :], axis=-1
    ).astype(jnp.int32)

    attn = pl.pallas_call(
        _attn_body,
        grid_spec=pltpu.PrefetchScalarGridSpec(
            num_scalar_prefetch=3,
            grid=(n, NUM_PAGES),
            in_specs=[
                pl.BlockSpec((1, N_HEADS, HEAD_DIM),
                             lambda i, j, bt, cl, ht: (i, 0, 0)),
                pl.BlockSpec((1, BLOCK_SIZE, HEAD_DIM),
                             lambda i, j, bt, cl, ht: (
                                 bt[i, jnp.minimum(
                                     j, (cl[i] - 1) // BLOCK_SIZE)], 0, 0)),
                pl.BlockSpec((N_TOK, HEAD_DIM),
                             lambda i, j, bt, cl, ht: (0, 0)),
                pl.BlockSpec((1, N_TOK), lambda i, j, bt, cl, ht: (0, 0)),
                pl.BlockSpec((N_HEADS, 1), lambda i, j, bt, cl, ht: (0, 0)),
            ],
            out_specs=pl.BlockSpec((1, N_HEADS, HEAD_DIM),
                                   lambda i, j, bt, cl, ht: (i, 0, 0)),
            scratch_shapes=[
                pltpu.VMEM((N_HEADS, HEAD_DIM), jnp.float32),
                pltpu.VMEM((N_HEADS, 1), jnp.float32),
                pltpu.VMEM((N_HEADS, 1), jnp.float32),
            ],
        ),
        out_shape=jax.ShapeDtypeStruct((n, N_HEADS, HEAD_DIM), jnp.float32),
        compiler_params=pltpu.CompilerParams(
            dimension_semantics=("arbitrary", "arbitrary")),
    )(bt32, context_lens.astype(jnp.int32), hit_flags,
      q, cache, new_kv, slot_mapping.reshape(1, n).astype(jnp.int32),
      attn_sink.reshape(N_HEADS, 1))

    o = pl.pallas_call(
        _out_body,
        out_shape=jax.ShapeDtypeStruct((n, DIM), jnp.float32),
    )(attn.reshape(n, QDIM), wo_a, wo_b)
    return o
